# baseline (speedup 1.0000x reference)
import functools

import jax
import jax.numpy as jnp
from jax import lax
from jax.experimental import pallas as pl
from jax.experimental.pallas import tpu as pltpu
from jax.experimental.pallas import tpu_sc as plsc

B, L, VOCAB, D = 1024, 20, 1000, 2048
W = D // 2
HALO = 32
WH = W + HALO
NC, NS = 2, 16
NW = NC * NS
BPW = B // NW


def _sc_encode(tokens, tab):
    mesh = plsc.VectorSubcoreMesh(core_axis_name="c", subcore_axis_name="s")

    @functools.partial(
        pl.kernel,
        out_type=jax.ShapeDtypeStruct((B, D), jnp.int32),
        mesh=mesh,
        compiler_params=pltpu.CompilerParams(
            use_tc_tiling_on_sc=False, needs_layout_passes=False),
        scratch_types=[
            pltpu.VMEM((BPW, L), jnp.int32),
            pltpu.VMEM((L, WH), jnp.int32),
            pltpu.VMEM((L, WH), jnp.int32),
            pltpu.VMEM((D,), jnp.int32),
            pltpu.VMEM((D,), jnp.int32),
            pltpu.SemaphoreType.DMA,
            pltpu.SemaphoreType.DMA,
            pltpu.SemaphoreType.DMA,
            pltpu.SemaphoreType.DMA,
        ],
    )
    def enc(tok_hbm, tab_hbm, out_hbm, tok_v, rows0, rows1,
            out0, out1, sem0, sem1, osem0, osem1):
        wid = lax.axis_index("s") * NC + lax.axis_index("c")
        base = wid * BPW
        pltpu.sync_copy(tok_hbm.at[pl.ds(base, BPW)], tok_v)

        rbufs, rsems = (rows0, rows1), (sem0, sem1)
        obufs, osems = (out0, out1), (osem0, osem1)
        gathers = [None, None]
        scatters = [None, None]

        ci = lax.iota(jnp.int32, 16)
        rsplat = [jnp.full((16,), l, jnp.int32) for l in range(L)]
        cil = [ci + (HALO - l) for l in range(L)]
        one = jnp.full((16,), 1, jnp.int32)
        mone = jnp.full((16,), -1, jnp.int32)

        gathers[0] = pltpu.async_copy(tab_hbm.at[tok_v.at[0]], rows0, sem0)
        for b in range(BPW):
            p = b % 2
            if b + 1 < BPW:
                gathers[1 - p] = pltpu.async_copy(
                    tab_hbm.at[tok_v.at[b + 1]], rbufs[1 - p], rsems[1 - p])
            gathers[p].wait()
            rows = rbufs[p]
            if scatters[p] is not None:
                scatters[p].wait()
            ov = obufs[p]

            @plsc.parallel_loop(0, W, step=16, unroll=2)
            def chunk(col, rows=rows, ov=ov):
                terms = [
                    plsc.bitcast(
                        plsc.load_gather(rows, [rsplat[l], cil[l] + col]),
                        jnp.bfloat16)
                    for l in range(L)
                ]
                while len(terms) > 1:
                    terms = [a + b for a, b in zip(terms[::2], terms[1::2])] \
                        + ([terms[-1]] if len(terms) % 2 else [])
                w = plsc.bitcast(terms[0], jnp.int32)
                lo = w & jnp.int32(0xFFFF)
                hi = (w >> 16) & jnp.int32(0xFFFF)
                ov[pl.ds(col, 16)] = jnp.where(
                    (lo > 0) & (lo < jnp.int32(0x8000)), one, mone)
                ov[pl.ds(col + W, 16)] = jnp.where(
                    (hi > 0) & (hi < jnp.int32(0x8000)), one, mone)

            scatters[p] = pltpu.async_copy(ov, out_hbm.at[base + b], osems[p])
        for s in scatters:
            if s is not None:
                s.wait()

    return enc(tokens, tab)


def kernel(tokens, item_mem):
    def pack(lo_vals, hi_vals):
        return (jnp.int32(0x3F803F80)
                ^ jnp.where(lo_vals < 0, jnp.int32(0x8000), jnp.int32(0))
                ^ jnp.where(hi_vals < 0, jnp.int32(-0x80000000),
                            jnp.int32(0)))

    tab = pack(
        jnp.concatenate([item_mem[:, D - HALO:], item_mem[:, :W]], axis=1),
        jnp.concatenate([item_mem[:, W - HALO:W], item_mem[:, W:]], axis=1))
    return _sc_encode(tokens, tab)


# speedup vs baseline: 1.1114x; 1.1114x over previous
import functools

import jax
import jax.numpy as jnp
from jax import lax
from jax.experimental import pallas as pl
from jax.experimental.pallas import tpu as pltpu
from jax.experimental.pallas import tpu_sc as plsc

B, L, VOCAB, D = 1024, 20, 1000, 2048
W = D // 2
HALO = 32
WH = W + HALO
NC, NS = 2, 16
NW = NC * NS
BPW = B // NW
NCHUNK = W // 16


def _sc_encode(tokens, tab):
    mesh = plsc.VectorSubcoreMesh(core_axis_name="c", subcore_axis_name="s")

    @functools.partial(
        pl.kernel,
        out_type=jax.ShapeDtypeStruct((B, D), jnp.int32),
        mesh=mesh,
        compiler_params=pltpu.CompilerParams(
            use_tc_tiling_on_sc=False, needs_layout_passes=False),
        scratch_types=[
            pltpu.VMEM((BPW, L), jnp.int32),
            pltpu.VMEM((L, WH), jnp.int32),
            pltpu.VMEM((L, WH), jnp.int32),
            pltpu.VMEM((D,), jnp.int32),
            pltpu.VMEM((D,), jnp.int32),
            pltpu.SemaphoreType.DMA,
            pltpu.SemaphoreType.DMA,
            pltpu.SemaphoreType.DMA,
            pltpu.SemaphoreType.DMA,
        ],
    )
    def enc(tok_hbm, tab_hbm, out_hbm, tok_v, rows0, rows1, out0, out1,
            sem0, sem1, osem0, osem1):
        wid = lax.axis_index("s") * NC + lax.axis_index("c")
        base = wid * BPW
        pltpu.sync_copy(tok_hbm.at[pl.ds(base, BPW)], tok_v)

        rbufs, rsems = (rows0, rows1), (sem0, sem1)
        obufs, osems = (out0, out1), (osem0, osem1)
        gathers = [None, None]
        scatters = [None, None]

        ci = lax.iota(jnp.int32, 16)
        rsplat = [jnp.full((16,), l, jnp.int32) for l in range(L)]
        cil = [ci + (HALO - l) for l in range(L)]
        one = jnp.full((16,), 1, jnp.int32)
        mone = jnp.full((16,), -1, jnp.int32)

        gathers[0] = pltpu.async_copy(tab_hbm.at[tok_v.at[0]], rows0, sem0)
        for b in range(BPW):
            p = b % 2
            if b + 1 < BPW:
                gathers[1 - p] = pltpu.async_copy(
                    tab_hbm.at[tok_v.at[b + 1]], rbufs[1 - p], rsems[1 - p])
            gathers[p].wait()
            rows = rbufs[p]
            if scatters[p] is not None:
                scatters[p].wait()
            ov = obufs[p]

            def chunk(col, rows, ov):
                terms = [
                    plsc.bitcast(
                        plsc.load_gather(rows, [rsplat[l], cil[l] + col]),
                        jnp.bfloat16)
                    for l in range(L)
                ]
                while len(terms) > 1:
                    terms = [a + b for a, b in zip(terms[::2], terms[1::2])] \
                        + ([terms[-1]] if len(terms) % 2 else [])
                lo, hi = plsc.unpack(
                    terms[0], format=plsc.PackFormat.INTERLEAVED)
                ov[pl.ds(col, 16)] = jnp.where(lo > 0.0, one, mone)
                ov[pl.ds(col + W, 16)] = jnp.where(hi > 0.0, one, mone)

            def jbody(jc, _, rows=rows, ov=ov):
                col = jc * 32
                chunk(col, rows, ov)
                chunk(col + 16, rows, ov)
                return 0

            lax.fori_loop(0, NCHUNK // 2, jbody, 0)
            scatters[p] = pltpu.async_copy(ov, out_hbm.at[base + b], osems[p])
        for s in scatters:
            if s is not None:
                s.wait()

    return enc(tokens, tab)


def kernel(tokens, item_mem):
    s = lax.bitcast_convert_type(item_mem, jnp.uint32)
    slo = jnp.concatenate([s[:, D - HALO:], s[:, :W]], axis=1)
    shi = jnp.concatenate([s[:, W - HALO:W], s[:, W:]], axis=1)
    word = (jnp.uint32(0x3F803F80)
            | ((slo >> 16) & jnp.uint32(0x8000))
            | (shi & jnp.uint32(0x80000000)))
    tab = lax.bitcast_convert_type(word, jnp.int32)
    return _sc_encode(tokens, tab)
